# Initial kernel scaffold; baseline (speedup 1.0000x reference)
import jax
import jax.numpy as jnp
from jax.experimental import pallas as pl


def kernel(x, edge_index, batch, W_l, b_l, W_r, b_r, att):
    raise NotImplementedError("write your pallas kernel here")


# speedup vs baseline: 1.2292x; 1.2292x over previous
import functools

import jax
import jax.numpy as jnp
from jax import lax
from jax.experimental import pallas as pl
from jax.experimental.pallas import tpu as pltpu

_K = 20
_RB = 64


def _scores_topk_body(x_ref, wl_ref, wr_ref, bl_ref, brc_ref, attr_ref,
                      attc_ref, atts_ref, attn_ref, idx_ref, n, d, k):
    b = pl.program_id(0)
    r = pl.program_id(1)

    xb = x_ref[0]
    xrows = x_ref[0, pl.ds(r * _RB, _RB), :]

    x_l = lax.dot_general(xrows, wl_ref[...],
                          (((1,), (1,)), ((), ())),
                          preferred_element_type=jnp.float32) + bl_ref[...]
    x_rT = lax.dot_general(wr_ref[...], xb,
                           (((1,), (1,)), ((), ())),
                           preferred_element_type=jnp.float32) + brc_ref[...]

    sL = jnp.sum(x_l * attr_ref[...], axis=1, keepdims=True)
    sR = jnp.sum(x_rT * attc_ref[...], axis=0, keepdims=True)
    acc = 0.6 * (sL + sR)

    for dd in range(d):
        lcol = x_l[:, dd:dd + 1]
        rrow = x_rT[dd:dd + 1, :]
        a4 = atts_ref[0, dd] * 0.4
        acc = acc + a4 * jnp.abs(lcol + rrow)

    finite = (acc * 0.0) == 0.0
    acc = jnp.where(finite, acc, 0.0)

    colids = lax.broadcasted_iota(jnp.int32, (_RB, n), 1)
    vals_list = []
    idx_list = []
    cur = acc
    for _t in range(k):
        m = jnp.max(cur, axis=1, keepdims=True)
        cand = jnp.where(cur == m, colids, n)
        j = jnp.min(cand, axis=1, keepdims=True)
        vals_list.append(m)
        idx_list.append(j)
        cur = jnp.where(colids == j, -jnp.inf, cur)

    vals = jnp.concatenate(vals_list, axis=1)
    idxs = jnp.concatenate(idx_list, axis=1)

    e = jnp.exp(vals - vals[:, 0:1])
    attn = e / jnp.sum(e, axis=1, keepdims=True)

    attn_ref[0] = attn
    idx_ref[0] = idxs + b * n


@functools.partial(jax.jit, static_argnames=())
def _feature_graph(x, W_l, b_l, W_r, b_r, att):
    b, n, in_ch = x.shape
    d = W_l.shape[0]
    k = min(_K, n)
    nr = n // _RB

    grid = (b, nr)
    attn_out, idx_out = pl.pallas_call(
        functools.partial(_scores_topk_body, n=n, d=d, k=k),
        grid=grid,
        in_specs=[
            pl.BlockSpec((1, n, in_ch), lambda bb, rr: (bb, 0, 0)),
            pl.BlockSpec((d, in_ch), lambda bb, rr: (0, 0)),
            pl.BlockSpec((d, in_ch), lambda bb, rr: (0, 0)),
            pl.BlockSpec((1, d), lambda bb, rr: (0, 0)),
            pl.BlockSpec((d, 1), lambda bb, rr: (0, 0)),
            pl.BlockSpec((1, d), lambda bb, rr: (0, 0)),
            pl.BlockSpec((d, 1), lambda bb, rr: (0, 0)),
            pl.BlockSpec(memory_space=pltpu.SMEM),
        ],
        out_specs=[
            pl.BlockSpec((1, _RB, k), lambda bb, rr: (bb, rr, 0)),
            pl.BlockSpec((1, _RB, k), lambda bb, rr: (bb, rr, 0)),
        ],
        out_shape=[
            jax.ShapeDtypeStruct((b, n, k), jnp.float32),
            jax.ShapeDtypeStruct((b, n, k), jnp.int32),
        ],
    )(x, W_l, W_r, b_l.reshape(1, d), b_r.reshape(d, 1),
      att.reshape(1, d), att.reshape(d, 1), att.reshape(1, d))
    return attn_out, idx_out


def kernel(x, edge_index, batch, W_l, b_l, W_r, b_r, att):
    b, n, _ = x.shape
    k = min(_K, n)
    attn_out, idx_out = _feature_graph(x, W_l, b_l, W_r, b_r, att)
    attention = attn_out.reshape(-1)
    index_j = idx_out.reshape(-1)
    offset = (jnp.arange(b) * n)[:, None]
    index_i = (jnp.tile(jnp.repeat(jnp.arange(n), k), (b,)).reshape(b, -1)
               + offset).reshape(-1)
    new_edge_index = jnp.stack([index_i, index_j], axis=0)
    return new_edge_index, attention


# speedup vs baseline: 2.4702x; 2.0096x over previous
import functools

import jax
import jax.numpy as jnp
from jax import lax
from jax.experimental import pallas as pl
from jax.experimental.pallas import tpu as pltpu

_K = 20
_RB = 128


def _scores_topk_body(x_ref, wl_ref, wr_ref, bl_ref, brc_ref,
                      attc_ref, atts_ref, attn_ref, idx_ref, n, d, k):
    b = pl.program_id(0)
    r = pl.program_id(1)

    xb = x_ref[0]
    xrows = x_ref[0, pl.ds(r * _RB, _RB), :]

    x_l = lax.dot_general(xrows, wl_ref[...],
                          (((1,), (1,)), ((), ())),
                          preferred_element_type=jnp.float32) + bl_ref[...]
    x_rT = lax.dot_general(wr_ref[...], xb,
                           (((1,), (1,)), ((), ())),
                           preferred_element_type=jnp.float32) + brc_ref[...]

    sR = jnp.sum(x_rT * attc_ref[...], axis=0, keepdims=True)

    acc0 = 0.6 * sR + jnp.zeros((_RB, n), jnp.float32)
    acc1 = jnp.zeros((_RB, n), jnp.float32)
    for dd in range(d):
        lcol = x_l[:, dd:dd + 1]
        rrow = x_rT[dd:dd + 1, :]
        a4 = atts_ref[0, dd] * 0.4
        if dd % 2 == 0:
            acc0 = acc0 + a4 * jnp.abs(lcol + rrow)
        else:
            acc1 = acc1 + a4 * jnp.abs(lcol + rrow)
    acc = acc0 + acc1

    finite = (acc * 0.0) == 0.0
    acc = jnp.where(finite, acc, 0.0)

    colids = lax.broadcasted_iota(jnp.int32, (_RB, n), 1).astype(jnp.float32)
    nf = jnp.float32(n)
    vals_list = []
    idx_list = []
    cur = acc
    for _t in range(k):
        m = jnp.max(cur, axis=1, keepdims=True)
        cand = jnp.where(cur == m, colids, nf)
        j = jnp.min(cand, axis=1, keepdims=True)
        vals_list.append(m)
        idx_list.append(j)
        cur = jnp.where(colids == j, -jnp.inf, cur)

    vals = jnp.concatenate(vals_list, axis=1)
    idxs = jnp.concatenate(idx_list, axis=1)

    e = jnp.exp(vals - vals[:, 0:1])
    attn = e / jnp.sum(e, axis=1, keepdims=True)

    attn_ref[0] = attn
    idx_ref[0] = idxs.astype(jnp.int32) + b * n


@jax.jit
def _feature_graph(x, W_l, b_l, W_r, b_r, att):
    b, n, in_ch = x.shape
    d = W_l.shape[0]
    k = min(_K, n)
    nr = n // _RB

    grid = (b, nr)
    attn_out, idx_out = pl.pallas_call(
        functools.partial(_scores_topk_body, n=n, d=d, k=k),
        grid=grid,
        in_specs=[
            pl.BlockSpec((1, n, in_ch), lambda bb, rr: (bb, 0, 0)),
            pl.BlockSpec((d, in_ch), lambda bb, rr: (0, 0)),
            pl.BlockSpec((d, in_ch), lambda bb, rr: (0, 0)),
            pl.BlockSpec((1, d), lambda bb, rr: (0, 0)),
            pl.BlockSpec((d, 1), lambda bb, rr: (0, 0)),
            pl.BlockSpec((d, 1), lambda bb, rr: (0, 0)),
            pl.BlockSpec(memory_space=pltpu.SMEM),
        ],
        out_specs=[
            pl.BlockSpec((1, _RB, k), lambda bb, rr: (bb, rr, 0)),
            pl.BlockSpec((1, _RB, k), lambda bb, rr: (bb, rr, 0)),
        ],
        out_shape=[
            jax.ShapeDtypeStruct((b, n, k), jnp.float32),
            jax.ShapeDtypeStruct((b, n, k), jnp.int32),
        ],
    )(x, W_l, W_r, b_l.reshape(1, d), b_r.reshape(d, 1),
      att.reshape(d, 1), att.reshape(1, d))
    return attn_out, idx_out


def kernel(x, edge_index, batch, W_l, b_l, W_r, b_r, att):
    b, n, _ = x.shape
    k = min(_K, n)
    attn_out, idx_out = _feature_graph(x, W_l, b_l, W_r, b_r, att)
    attention = attn_out.reshape(-1)
    index_j = idx_out.reshape(-1)
    offset = (jnp.arange(b) * n)[:, None]
    index_i = (jnp.tile(jnp.repeat(jnp.arange(n), k), (b,)).reshape(b, -1)
               + offset).reshape(-1)
    new_edge_index = jnp.stack([index_i, index_j], axis=0)
    return new_edge_index, attention
